# baseline (speedup 1.0000x reference)
import functools

import jax
import jax.numpy as jnp
from jax import lax
from jax.experimental import pallas as pl
from jax.experimental.pallas import tpu as pltpu
from jax.experimental.pallas import tpu_sc as plsc

NUM_ROWS = 1000000
DIM = 64
B_TOTAL = 4096 * 200
NC, NS = 2, 16
NW = NC * NS
B_PER_W = B_TOTAL // NW
CHUNK = 128
N_CHUNKS = B_PER_W // CHUNK

_mesh = plsc.VectorSubcoreMesh(core_axis_name="c", subcore_axis_name="s")


@functools.partial(
    pl.kernel,
    mesh=_mesh,
    out_type=jax.ShapeDtypeStruct((B_TOTAL, DIM), jnp.float32),
    scratch_types=[
        pltpu.VMEM((N_CHUNKS, CHUNK), jnp.int32),
        pltpu.VMEM((CHUNK, DIM), jnp.float32),
        pltpu.SemaphoreType.DMA,
    ],
    compiler_params=pltpu.CompilerParams(use_tc_tiling_on_sc=False),
)
def _gather_kernel(idx_hbm, table_hbm, out_hbm, idx_v, rows_v, sem):
    wid = lax.axis_index("s") * NC + lax.axis_index("c")
    base = wid * B_PER_W
    pltpu.sync_copy(idx_hbm.at[wid], idx_v)

    def body(j, carry):
        pltpu.async_copy(table_hbm.at[idx_v.at[j]], rows_v, sem).wait()
        pltpu.sync_copy(rows_v, out_hbm.at[pl.ds(base + j * CHUNK, CHUNK)])
        return carry

    lax.fori_loop(0, N_CHUNKS, body, 0)


def kernel(input_ids, weight):
    flat = input_ids.reshape(NW, N_CHUNKS, CHUNK).astype(jnp.int32)
    out = _gather_kernel(flat, weight)
    return out.reshape(input_ids.shape + (DIM,))


# speedup vs baseline: 1.1172x; 1.1172x over previous
import functools

import jax
import jax.numpy as jnp
from jax import lax
from jax.experimental import pallas as pl
from jax.experimental.pallas import tpu as pltpu
from jax.experimental.pallas import tpu_sc as plsc

NUM_ROWS = 1000000
DIM = 64
B_TOTAL = 4096 * 200
NC, NS = 2, 16
NW = NC * NS
B_PER_W = B_TOTAL // NW
CHUNK = 128
N_CHUNKS = B_PER_W // CHUNK
NBUF = 8
G = 6
S = NBUF - G
N_GROUPS = N_CHUNKS // NBUF

_mesh = plsc.VectorSubcoreMesh(core_axis_name="c", subcore_axis_name="s")


@functools.partial(
    pl.kernel,
    mesh=_mesh,
    out_type=jax.ShapeDtypeStruct((B_TOTAL, DIM), jnp.float32),
    scratch_types=[
        pltpu.VMEM((N_CHUNKS, CHUNK), jnp.int32),
        pltpu.VMEM((NBUF, CHUNK, DIM), jnp.float32),
        pltpu.SemaphoreType.DMA,
        pltpu.SemaphoreType.DMA,
    ],
    compiler_params=pltpu.CompilerParams(use_tc_tiling_on_sc=False),
)
def _gather_kernel(idx_hbm, table_hbm, out_hbm, idx_v, rows_v, gsem, ssem):
    wid = lax.axis_index("s") * NC + lax.axis_index("c")
    base = wid * B_PER_W
    pltpu.sync_copy(idx_hbm.at[wid], idx_v)

    def gather(chunk, buf):
        pltpu.async_copy(table_hbm.at[idx_v.at[chunk]], rows_v.at[buf], gsem)

    def store(chunk, buf):
        pltpu.async_copy(
            rows_v.at[buf], out_hbm.at[pl.ds(base + chunk * CHUNK, CHUNK)], ssem
        )

    def wait_gather(buf):
        pltpu.make_async_copy(
            out_hbm.at[pl.ds(base, CHUNK)], rows_v.at[buf], gsem
        ).wait()

    def wait_store(buf):
        pltpu.make_async_copy(
            rows_v.at[buf], out_hbm.at[pl.ds(base, CHUNK)], ssem
        ).wait()

    for b in range(G):
        gather(b, b)

    for b in range(NBUF):
        wait_gather(b)
        store(b, b)
        if b >= S:
            wait_store(b)
        gather(b + G, (b + G) % NBUF)

    def group(g, carry):
        j0 = g * NBUF
        for b in range(NBUF):
            j = j0 + b
            wait_gather(b)
            store(j, b)
            wait_store(b)
            gather(j + G, (b + G) % NBUF)
        return carry

    lax.fori_loop(1, N_GROUPS - 1, group, 0)

    j0 = (N_GROUPS - 1) * NBUF
    for b in range(NBUF):
        j = j0 + b
        wait_gather(b)
        store(j, b)
        wait_store(b)
        if j + G < N_CHUNKS:
            gather(j + G, (b + G) % NBUF)

    for b in range(S):
        wait_store(b)


def kernel(input_ids, weight):
    flat = input_ids.reshape(NW, N_CHUNKS, CHUNK).astype(jnp.int32)
    out = _gather_kernel(flat, weight)
    return out.reshape(input_ids.shape + (DIM,))
